# baseline (speedup 1.0000x reference)
import math

import jax
import jax.numpy as jnp
from jax.experimental import pallas as pl
from jax.experimental.pallas import tpu as pltpu

_N = 1024
_NFEAT = 128
_HIDDEN = 64
_NCLASS = 40
_NUM_LAYERS = 2
_ALPHA = 0.1
_THETA = 0.5
_K = 8
_BLK = _N // _K


def _split_cat(hf):
    hb = hf.astype(jnp.bfloat16)
    hr = (hf - hb.astype(jnp.float32)).astype(jnp.bfloat16)
    return jnp.concatenate([hb, hr], axis=1)


def _mm(a, b):
    return jax.lax.dot_general(a, b, (((1,), (0,)), ((), ())),
                               precision=jax.lax.Precision.HIGHEST)


def _magg(maskb, hcat):
    o = jax.lax.dot_general(maskb, hcat, (((0,), (0,)), ((), ())),
                            preferred_element_type=jnp.float32)
    return o


def _gcn2_fwd(x_ref, adj_ref, w0_ref, b0_ref, w1_ref, b1_ref, cw_ref,
              out_ref, hcat_ref, h0_ref, maskb_ref, agg_ref, z_ref):
    i = pl.program_id(0)

    @pl.when(i == 0)
    def _prologue():
        h = jnp.maximum(_mm(x_ref[...], w0_ref[...]) + b0_ref[...], 0.0)
        h0_ref[...] = h
        hcat_ref[...] = _split_cat(h)
        agg_ref[...] = jnp.zeros_like(agg_ref)
        z_ref[...] = jnp.zeros_like(z_ref)

    adj = adj_ref[...]
    nz = (adj != 0.0)
    maskb = nz.astype(jnp.bfloat16)
    maskb_ref[pl.ds(i * _BLK, _BLK), :] = maskb
    zeros_here = jnp.float32(_BLK * _N) - jnp.sum(nz.astype(jnp.float32))
    z_ref[...] = z_ref[...] + zeros_here
    hc = hcat_ref[pl.ds(i * _BLK, _BLK), :]
    agg_ref[...] = agg_ref[...] + _magg(maskb, hc)

    @pl.when(i == _K - 1)
    def _epilogue():
        z = z_ref[0, 0]
        x0 = h0_ref[...]
        row_is0 = jax.lax.broadcasted_iota(jnp.int32, (_N, 1), 0) == 0

        def layer_update(agg2, h_prev, layer):
            beta = math.log(_THETA / (layer + 1) + 1.0)
            agg2 = agg2 + jnp.where(row_is0, z * h_prev[0:1, :], 0.0)
            out = agg2 * (1.0 - _ALPHA) + _ALPHA * x0
            out = (1.0 - beta) * out + beta * _mm(out, cw_ref[layer])
            return jnp.maximum(out, 0.0)

        acc = agg_ref[...]
        h1 = layer_update(acc[:, :_HIDDEN] + acc[:, _HIDDEN:], h0_ref[...], 0)
        o2 = _magg(maskb_ref[...], _split_cat(h1))
        h2 = layer_update(o2[:, :_HIDDEN] + o2[:, _HIDDEN:], h1, 1)

        logits = _mm(h2, w1_ref[...]) + b1_ref[...]
        m = jnp.max(logits, axis=-1, keepdims=True)
        s = logits - m
        lse = jnp.log(jnp.sum(jnp.exp(s), axis=-1, keepdims=True))
        out_ref[...] = s - lse


def kernel(x, adj_t, lin0_w, lin0_b, lin1_w, lin1_b, conv_w):
    b0 = lin0_b.reshape(1, _HIDDEN)
    b1 = lin1_b.reshape(1, _NCLASS)
    full = lambda *shape: pl.BlockSpec(shape, lambda i: tuple(0 for _ in shape))
    return pl.pallas_call(
        _gcn2_fwd,
        grid=(_K,),
        in_specs=[
            full(_N, _NFEAT),
            pl.BlockSpec((_BLK, _N), lambda i: (i, 0)),
            full(_NFEAT, _HIDDEN),
            full(1, _HIDDEN),
            full(_HIDDEN, _NCLASS),
            full(1, _NCLASS),
            full(_NUM_LAYERS, _HIDDEN, _HIDDEN),
        ],
        out_specs=full(_N, _NCLASS),
        out_shape=jax.ShapeDtypeStruct((_N, _NCLASS), jnp.float32),
        scratch_shapes=[
            pltpu.VMEM((_N, 2 * _HIDDEN), jnp.bfloat16),
            pltpu.VMEM((_N, _HIDDEN), jnp.float32),
            pltpu.VMEM((_N, _N), jnp.bfloat16),
            pltpu.VMEM((_N, 2 * _HIDDEN), jnp.float32),
            pltpu.VMEM((1, 1), jnp.float32),
        ],
    )(x, adj_t, lin0_w, b0, lin1_w, b1, conv_w)
